# baseline (speedup 1.0000x reference)
import functools

import jax
import jax.numpy as jnp
from jax import lax
from jax.experimental import pallas as pl
from jax.experimental.pallas import tpu as pltpu
from jax.experimental.pallas import tpu_sc as plsc

N_TOK = 16 * 576
K = 8192
D = 256

TN = 512
TK = 1024
NB = N_TOK // TN
KB = K // TK
NSTEP = NB * KB + 1

SC_NC = 2
SC_NS = 16
SC_NW = SC_NC * SC_NS
BPW = N_TOK // SC_NW


def _argmin_body(xsq_ref, x_ref, e_ref, esq_ref, out_ref,
                 x2_s, buf0, buf1, best_val, best_idx):
    t = pl.program_id(0)

    @pl.when(t % KB == 0)
    def _fresh_x():
        x2_s[...] = x_ref[...] + x_ref[...]

    def step(wbuf, rbuf):
        e_tile = e_ref[pl.ds((t % KB) * TK, TK), :]
        wbuf[...] = lax.dot_general(
            x2_s[...], e_tile, (((1,), (1,)), ((), ())),
            preferred_element_type=jnp.float32)
        s = (xsq_ref[...] - rbuf[...]) + esq_ref[...]
        m = jnp.min(s, axis=1, keepdims=True)
        iota = lax.broadcasted_iota(jnp.int32, (TN, TK), 1)
        loc = jnp.min(jnp.where(s == m, iota, TK), axis=1, keepdims=True)
        cand = loc + ((t - 1) % KB) * TK
        first = (t - 1) % KB == 0
        prev_v = jnp.where(first, jnp.inf, best_val[...])
        prev_i = best_idx[...]
        m = jnp.where(t > 0, m, jnp.inf)
        better = m < prev_v
        best_val[...] = jnp.where(better, m, prev_v)
        best_idx[...] = jnp.where(better, cand, prev_i)

    @pl.when(t % 2 == 0)
    def _even():
        step(buf0, buf1)

    @pl.when(t % 2 == 1)
    def _odd():
        step(buf1, buf0)

    @pl.when((t > 0) & (t % KB == 0))
    def _emit():
        out_ref[...] = best_idx[...]


_argmin_call = pl.pallas_call(
    _argmin_body,
    grid=(NSTEP,),
    in_specs=[
        pl.BlockSpec((TN, 1), lambda t: (jnp.clip((t - 1) // KB, 0, NB - 1), 0)),
        pl.BlockSpec((TN, D), lambda t: (jnp.minimum(t // KB, NB - 1), 0)),
        pl.BlockSpec((K, D), lambda t: (0, 0)),
        pl.BlockSpec((1, TK), lambda t: (0, jnp.maximum(t - 1, 0) % KB)),
    ],
    out_specs=pl.BlockSpec((TN, 1), lambda t: (jnp.clip((t - 1) // KB, 0, NB - 1), 0)),
    out_shape=jax.ShapeDtypeStruct((N_TOK, 1), jnp.int32),
    scratch_shapes=[
        pltpu.VMEM((TN, D), jnp.float32),
        pltpu.VMEM((TN, TK), jnp.float32),
        pltpu.VMEM((TN, TK), jnp.float32),
        pltpu.VMEM((TN, 1), jnp.float32),
        pltpu.VMEM((TN, 1), jnp.int32),
    ],
)


@functools.lru_cache(maxsize=1)
def _sc_gather():
    mesh = plsc.VectorSubcoreMesh(
        core_axis_name="c", subcore_axis_name="s",
        num_cores=SC_NC, num_subcores=SC_NS)

    @functools.partial(
        pl.kernel,
        mesh=mesh,
        out_type=jax.ShapeDtypeStruct((N_TOK, D), jnp.float32),
        scratch_types=[
            pltpu.VMEM((BPW,), jnp.int32),
            pltpu.VMEM((BPW, D), jnp.float32),
            pltpu.SemaphoreType.DMA,
        ],
    )
    def gather(table_hbm, idx_hbm, out_hbm, idx_v, rows_v, sem):
        wid = lax.axis_index("s") * SC_NC + lax.axis_index("c")
        base = wid * BPW
        pltpu.sync_copy(idx_hbm.at[pl.ds(base, BPW)], idx_v)
        pltpu.async_copy(table_hbm.at[idx_v], rows_v, sem).wait()
        pltpu.sync_copy(rows_v, out_hbm.at[pl.ds(base, BPW)])

    return gather


def kernel(x, embed):
    flatten = x.reshape(N_TOK, D)
    table = embed[0]
    x_sq = jnp.sum(flatten ** 2, axis=-1, keepdims=True)
    e_sq = jnp.sum(embed ** 2, axis=-1)
    idx = _argmin_call(x_sq, flatten, table, e_sq).reshape(N_TOK)
    quantize = _sc_gather()(table, idx)
    return quantize.reshape(x.shape), idx.reshape(x.shape[:-1])


# speedup vs baseline: 1.0570x; 1.0570x over previous
import functools

import jax
import jax.numpy as jnp
from jax import lax
from jax.experimental import pallas as pl
from jax.experimental.pallas import tpu as pltpu
from jax.experimental.pallas import tpu_sc as plsc

N_TOK = 16 * 576
K = 8192
D = 256

TN = 512
TK = 2048
NB = N_TOK // TN
KB = K // TK
NSTEP = NB * KB + 1

SC_NC = 2
SC_NS = 16
SC_NW = SC_NC * SC_NS
BPW = N_TOK // SC_NW


def _argmin_body(xsq_ref, x_ref, e_ref, esq_ref, out_ref,
                 x2_s, buf0, buf1, best_val, best_idx):
    t = pl.program_id(0)

    @pl.when(t % KB == 0)
    def _fresh_x():
        x2_s[...] = x_ref[...] + x_ref[...]

    def step(wbuf, rbuf):
        e_tile = e_ref[pl.ds((t % KB) * TK, TK), :]
        wbuf[...] = lax.dot_general(
            x2_s[...], e_tile, (((1,), (1,)), ((), ())),
            preferred_element_type=jnp.float32)
        s = (xsq_ref[...] - rbuf[...]) + esq_ref[...]
        m = jnp.min(s, axis=1, keepdims=True)
        iota = lax.broadcasted_iota(jnp.int32, (TN, TK), 1)
        loc = jnp.min(jnp.where(s == m, iota, TK), axis=1, keepdims=True)
        cand = loc + ((t - 1) % KB) * TK
        first = (t - 1) % KB == 0
        prev_v = jnp.where(first, jnp.inf, best_val[...])
        prev_i = best_idx[...]
        m = jnp.where(t > 0, m, jnp.inf)
        better = m < prev_v
        best_val[...] = jnp.where(better, m, prev_v)
        best_idx[...] = jnp.where(better, cand, prev_i)

    @pl.when(t % 2 == 0)
    def _even():
        step(buf0, buf1)

    @pl.when(t % 2 == 1)
    def _odd():
        step(buf1, buf0)

    @pl.when((t > 0) & (t % KB == 0))
    def _emit():
        out_ref[...] = best_idx[...]


_argmin_call = pl.pallas_call(
    _argmin_body,
    grid=(NSTEP,),
    in_specs=[
        pl.BlockSpec((TN, 1), lambda t: (jnp.clip((t - 1) // KB, 0, NB - 1), 0)),
        pl.BlockSpec((TN, D), lambda t: (jnp.minimum(t // KB, NB - 1), 0)),
        pl.BlockSpec((K, D), lambda t: (0, 0)),
        pl.BlockSpec((1, TK), lambda t: (0, jnp.maximum(t - 1, 0) % KB)),
    ],
    out_specs=pl.BlockSpec((TN, 1), lambda t: (jnp.clip((t - 1) // KB, 0, NB - 1), 0)),
    out_shape=jax.ShapeDtypeStruct((N_TOK, 1), jnp.int32),
    scratch_shapes=[
        pltpu.VMEM((TN, D), jnp.float32),
        pltpu.VMEM((TN, TK), jnp.float32),
        pltpu.VMEM((TN, TK), jnp.float32),
        pltpu.VMEM((TN, 1), jnp.float32),
        pltpu.VMEM((TN, 1), jnp.int32),
    ],
)


@functools.lru_cache(maxsize=1)
def _sc_gather():
    mesh = plsc.VectorSubcoreMesh(
        core_axis_name="c", subcore_axis_name="s",
        num_cores=SC_NC, num_subcores=SC_NS)

    @functools.partial(
        pl.kernel,
        mesh=mesh,
        out_type=jax.ShapeDtypeStruct((N_TOK, D), jnp.float32),
        scratch_types=[
            pltpu.VMEM((BPW,), jnp.int32),
            pltpu.VMEM((BPW, D), jnp.float32),
            pltpu.SemaphoreType.DMA,
        ],
    )
    def gather(table_hbm, idx_hbm, out_hbm, idx_v, rows_v, sem):
        wid = lax.axis_index("s") * SC_NC + lax.axis_index("c")
        base = wid * BPW
        pltpu.sync_copy(idx_hbm.at[pl.ds(base, BPW)], idx_v)
        pltpu.async_copy(table_hbm.at[idx_v], rows_v, sem).wait()
        pltpu.sync_copy(rows_v, out_hbm.at[pl.ds(base, BPW)])

    return gather


def kernel(x, embed):
    flatten = x.reshape(N_TOK, D)
    table = embed[0]
    x_sq = jnp.sum(flatten ** 2, axis=-1, keepdims=True)
    e_sq = jnp.sum(embed ** 2, axis=-1)
    idx = _argmin_call(x_sq, flatten, table, e_sq).reshape(N_TOK)
    quantize = _sc_gather()(table, idx)
    return quantize.reshape(x.shape), idx.reshape(x.shape[:-1])


# speedup vs baseline: 1.1741x; 1.1108x over previous
import functools

import jax
import jax.numpy as jnp
from jax import lax
from jax.experimental import pallas as pl
from jax.experimental.pallas import tpu as pltpu
from jax.experimental.pallas import tpu_sc as plsc

N_TOK = 16 * 576
K = 8192
D = 256

TN = 512
TK = 2048
NB = N_TOK // TN
KB = K // TK
NSTEP = NB * KB + 1

SC_NC = 2
SC_NS = 16
SC_NW = SC_NC * SC_NS
BPW = N_TOK // SC_NW


def _argmin_body(xsq_ref, x_ref, e_ref, esq_ref, out_ref,
                 x2_s, buf0, buf1, best_val, best_idx):
    t = pl.program_id(0)

    @pl.when(t % KB == 0)
    def _fresh_x():
        x2_s[...] = x_ref[...] + x_ref[...]

    def step(wbuf, rbuf):
        e_tile = e_ref[pl.ds((t % KB) * TK, TK), :]
        wbuf[...] = lax.dot_general(
            x2_s[...], e_tile, (((1,), (1,)), ((), ())),
            preferred_element_type=jnp.float32)
        xsq = xsq_ref[...]
        acc_v = (xsq - rbuf[:, 0:128]) + esq_ref[:, 0:128]
        acc_c = jnp.zeros((TN, 128), jnp.int32)
        for c in range(1, TK // 128):
            sc = (xsq - rbuf[:, c * 128:(c + 1) * 128]) \
                + esq_ref[:, c * 128:(c + 1) * 128]
            mask = sc < acc_v
            acc_v = jnp.where(mask, sc, acc_v)
            acc_c = jnp.where(mask, c, acc_c)
        m = jnp.min(acc_v, axis=1, keepdims=True)
        lane = lax.broadcasted_iota(jnp.int32, (TN, 128), 1)
        gidx = acc_c * 128 + lane
        loc = jnp.min(jnp.where(acc_v == m, gidx, TK), axis=1, keepdims=True)
        cand = loc + ((t - 1) % KB) * TK
        first = (t - 1) % KB == 0
        prev_v = jnp.where(first, jnp.inf, best_val[...])
        prev_i = best_idx[...]
        m = jnp.where(t > 0, m, jnp.inf)
        better = m < prev_v
        best_val[...] = jnp.where(better, m, prev_v)
        best_idx[...] = jnp.where(better, cand, prev_i)

    @pl.when(t % 2 == 0)
    def _even():
        step(buf0, buf1)

    @pl.when(t % 2 == 1)
    def _odd():
        step(buf1, buf0)

    @pl.when((t > 0) & (t % KB == 0))
    def _emit():
        out_ref[...] = best_idx[...]


_argmin_call = pl.pallas_call(
    _argmin_body,
    grid=(NSTEP,),
    in_specs=[
        pl.BlockSpec((TN, 1), lambda t: (jnp.clip((t - 1) // KB, 0, NB - 1), 0)),
        pl.BlockSpec((TN, D), lambda t: (jnp.minimum(t // KB, NB - 1), 0)),
        pl.BlockSpec((K, D), lambda t: (0, 0)),
        pl.BlockSpec((1, TK), lambda t: (0, jnp.maximum(t - 1, 0) % KB)),
    ],
    out_specs=pl.BlockSpec((TN, 1), lambda t: (jnp.clip((t - 1) // KB, 0, NB - 1), 0)),
    out_shape=jax.ShapeDtypeStruct((N_TOK, 1), jnp.int32),
    scratch_shapes=[
        pltpu.VMEM((TN, D), jnp.float32),
        pltpu.VMEM((TN, TK), jnp.float32),
        pltpu.VMEM((TN, TK), jnp.float32),
        pltpu.VMEM((TN, 1), jnp.float32),
        pltpu.VMEM((TN, 1), jnp.int32),
    ],
)


@functools.lru_cache(maxsize=1)
def _sc_gather():
    mesh = plsc.VectorSubcoreMesh(
        core_axis_name="c", subcore_axis_name="s",
        num_cores=SC_NC, num_subcores=SC_NS)

    @functools.partial(
        pl.kernel,
        mesh=mesh,
        out_type=jax.ShapeDtypeStruct((N_TOK, D), jnp.float32),
        scratch_types=[
            pltpu.VMEM((BPW,), jnp.int32),
            pltpu.VMEM((BPW, D), jnp.float32),
            pltpu.SemaphoreType.DMA,
        ],
    )
    def gather(table_hbm, idx_hbm, out_hbm, idx_v, rows_v, sem):
        wid = lax.axis_index("s") * SC_NC + lax.axis_index("c")
        base = wid * BPW
        pltpu.sync_copy(idx_hbm.at[pl.ds(base, BPW)], idx_v)
        pltpu.async_copy(table_hbm.at[idx_v], rows_v, sem).wait()
        pltpu.sync_copy(rows_v, out_hbm.at[pl.ds(base, BPW)])

    return gather


def kernel(x, embed):
    flatten = x.reshape(N_TOK, D)
    table = embed[0]
    x_sq = jnp.sum(flatten ** 2, axis=-1, keepdims=True)
    e_sq = jnp.sum(embed ** 2, axis=-1)
    idx = _argmin_call(x_sq, flatten, table, e_sq).reshape(N_TOK)
    quantize = _sc_gather()(table, idx)
    return quantize.reshape(x.shape), idx.reshape(x.shape[:-1])


# speedup vs baseline: 1.2804x; 1.0905x over previous
import functools

import jax
import jax.numpy as jnp
from jax import lax
from jax.experimental import pallas as pl
from jax.experimental.pallas import tpu as pltpu
from jax.experimental.pallas import tpu_sc as plsc

N_TOK = 16 * 576
K = 8192
D = 256

TN = 1024
TK = 2048
NB = N_TOK // TN
KB = K // TK
NSTEP = NB * KB + 1

SC_NC = 2
SC_NS = 16
SC_NW = SC_NC * SC_NS
BPW = N_TOK // SC_NW


def _argmin_body(xsq_ref, x_ref, e_ref, esq_ref, out_ref,
                 x2_s, buf0, buf1, best_val, best_idx):
    t = pl.program_id(0)

    @pl.when(t % KB == 0)
    def _fresh_x():
        x2_s[...] = x_ref[...] + x_ref[...]

    def step(wbuf, rbuf):
        e_tile = e_ref[pl.ds((t % KB) * TK, TK), :]
        wbuf[...] = lax.dot_general(
            x2_s[...], e_tile, (((1,), (1,)), ((), ())),
            preferred_element_type=jnp.float32)
        xsq = xsq_ref[...]
        acc_v = (xsq - rbuf[:, 0:128]) + esq_ref[:, 0:128]
        acc_c = jnp.zeros((TN, 128), jnp.int32)
        for c in range(1, TK // 128):
            sc = (xsq - rbuf[:, c * 128:(c + 1) * 128]) \
                + esq_ref[:, c * 128:(c + 1) * 128]
            mask = sc < acc_v
            acc_v = jnp.where(mask, sc, acc_v)
            acc_c = jnp.where(mask, c, acc_c)
        m = jnp.min(acc_v, axis=1, keepdims=True)
        lane = lax.broadcasted_iota(jnp.int32, (TN, 128), 1)
        gidx = acc_c * 128 + lane
        loc = jnp.min(jnp.where(acc_v == m, gidx, TK), axis=1, keepdims=True)
        cand = loc + ((t - 1) % KB) * TK
        first = (t - 1) % KB == 0
        prev_v = jnp.where(first, jnp.inf, best_val[...])
        prev_i = best_idx[...]
        m = jnp.where(t > 0, m, jnp.inf)
        better = m < prev_v
        best_val[...] = jnp.where(better, m, prev_v)
        best_idx[...] = jnp.where(better, cand, prev_i)

    @pl.when(t % 2 == 0)
    def _even():
        step(buf0, buf1)

    @pl.when(t % 2 == 1)
    def _odd():
        step(buf1, buf0)

    @pl.when((t > 0) & (t % KB == 0))
    def _emit():
        out_ref[...] = best_idx[...]


_argmin_call = pl.pallas_call(
    _argmin_body,
    grid=(NSTEP,),
    in_specs=[
        pl.BlockSpec((TN, 1), lambda t: (jnp.clip((t - 1) // KB, 0, NB - 1), 0)),
        pl.BlockSpec((TN, D), lambda t: (jnp.minimum(t // KB, NB - 1), 0)),
        pl.BlockSpec((K, D), lambda t: (0, 0)),
        pl.BlockSpec((1, TK), lambda t: (0, jnp.maximum(t - 1, 0) % KB)),
    ],
    out_specs=pl.BlockSpec((TN, 1), lambda t: (jnp.clip((t - 1) // KB, 0, NB - 1), 0)),
    out_shape=jax.ShapeDtypeStruct((N_TOK, 1), jnp.int32),
    scratch_shapes=[
        pltpu.VMEM((TN, D), jnp.float32),
        pltpu.VMEM((TN, TK), jnp.float32),
        pltpu.VMEM((TN, TK), jnp.float32),
        pltpu.VMEM((TN, 1), jnp.float32),
        pltpu.VMEM((TN, 1), jnp.int32),
    ],
)


@functools.lru_cache(maxsize=1)
def _sc_gather():
    mesh = plsc.VectorSubcoreMesh(
        core_axis_name="c", subcore_axis_name="s",
        num_cores=SC_NC, num_subcores=SC_NS)

    @functools.partial(
        pl.kernel,
        mesh=mesh,
        out_type=jax.ShapeDtypeStruct((N_TOK, D), jnp.float32),
        scratch_types=[
            pltpu.VMEM((BPW,), jnp.int32),
            pltpu.VMEM((BPW, D), jnp.float32),
            pltpu.SemaphoreType.DMA,
        ],
    )
    def gather(table_hbm, idx_hbm, out_hbm, idx_v, rows_v, sem):
        wid = lax.axis_index("s") * SC_NC + lax.axis_index("c")
        base = wid * BPW
        pltpu.sync_copy(idx_hbm.at[pl.ds(base, BPW)], idx_v)
        pltpu.async_copy(table_hbm.at[idx_v], rows_v, sem).wait()
        pltpu.sync_copy(rows_v, out_hbm.at[pl.ds(base, BPW)])

    return gather


def kernel(x, embed):
    flatten = x.reshape(N_TOK, D)
    table = embed[0]
    x_sq = jnp.sum(flatten ** 2, axis=-1, keepdims=True)
    e_sq = jnp.sum(embed ** 2, axis=-1)
    idx = _argmin_call(x_sq, flatten, table, e_sq).reshape(N_TOK)
    quantize = _sc_gather()(table, idx)
    return quantize.reshape(x.shape), idx.reshape(x.shape[:-1])


# speedup vs baseline: 1.3174x; 1.0289x over previous
import functools

import jax
import jax.numpy as jnp
from jax import lax
from jax.experimental import pallas as pl
from jax.experimental.pallas import tpu as pltpu
from jax.experimental.pallas import tpu_sc as plsc

N_TOK = 16 * 576
K = 8192
D = 256

TN = 1536
TK = 2048
NB = N_TOK // TN
KB = K // TK
NSTEP = NB * KB + 1

SC_NC = 2
SC_NS = 16
SC_NW = SC_NC * SC_NS
BPW = N_TOK // SC_NW


def _argmin_body(xsq_ref, x_ref, e_ref, esq_ref, out_ref,
                 x2_s, buf0, buf1, best_val, best_idx):
    t = pl.program_id(0)

    @pl.when(t % KB == 0)
    def _fresh_x():
        x2_s[...] = x_ref[...] + x_ref[...]

    def step(wbuf, rbuf):
        e_tile = e_ref[pl.ds((t % KB) * TK, TK), :]
        wbuf[...] = lax.dot_general(
            x2_s[...], e_tile, (((1,), (1,)), ((), ())),
            preferred_element_type=jnp.float32)
        xsq = xsq_ref[...]
        acc_v = (xsq - rbuf[:, 0:128]) + esq_ref[:, 0:128]
        acc_c = jnp.zeros((TN, 128), jnp.int32)
        for c in range(1, TK // 128):
            sc = (xsq - rbuf[:, c * 128:(c + 1) * 128]) \
                + esq_ref[:, c * 128:(c + 1) * 128]
            mask = sc < acc_v
            acc_v = jnp.where(mask, sc, acc_v)
            acc_c = jnp.where(mask, c, acc_c)
        m = jnp.min(acc_v, axis=1, keepdims=True)
        lane = lax.broadcasted_iota(jnp.int32, (TN, 128), 1)
        gidx = acc_c * 128 + lane
        loc = jnp.min(jnp.where(acc_v == m, gidx, TK), axis=1, keepdims=True)
        cand = loc + ((t - 1) % KB) * TK
        first = (t - 1) % KB == 0
        prev_v = jnp.where(first, jnp.inf, best_val[...])
        prev_i = best_idx[...]
        m = jnp.where(t > 0, m, jnp.inf)
        better = m < prev_v
        best_val[...] = jnp.where(better, m, prev_v)
        best_idx[...] = jnp.where(better, cand, prev_i)

    @pl.when(t % 2 == 0)
    def _even():
        step(buf0, buf1)

    @pl.when(t % 2 == 1)
    def _odd():
        step(buf1, buf0)

    @pl.when((t > 0) & (t % KB == 0))
    def _emit():
        out_ref[...] = best_idx[...]


_argmin_call = pl.pallas_call(
    _argmin_body,
    grid=(NSTEP,),
    in_specs=[
        pl.BlockSpec((TN, 1), lambda t: (jnp.clip((t - 1) // KB, 0, NB - 1), 0)),
        pl.BlockSpec((TN, D), lambda t: (jnp.minimum(t // KB, NB - 1), 0)),
        pl.BlockSpec((K, D), lambda t: (0, 0)),
        pl.BlockSpec((1, TK), lambda t: (0, jnp.maximum(t - 1, 0) % KB)),
    ],
    out_specs=pl.BlockSpec((TN, 1), lambda t: (jnp.clip((t - 1) // KB, 0, NB - 1), 0)),
    out_shape=jax.ShapeDtypeStruct((N_TOK, 1), jnp.int32),
    scratch_shapes=[
        pltpu.VMEM((TN, D), jnp.float32),
        pltpu.VMEM((TN, TK), jnp.float32),
        pltpu.VMEM((TN, TK), jnp.float32),
        pltpu.VMEM((TN, 1), jnp.float32),
        pltpu.VMEM((TN, 1), jnp.int32),
    ],
)


@functools.lru_cache(maxsize=1)
def _sc_gather():
    mesh = plsc.VectorSubcoreMesh(
        core_axis_name="c", subcore_axis_name="s",
        num_cores=SC_NC, num_subcores=SC_NS)

    @functools.partial(
        pl.kernel,
        mesh=mesh,
        out_type=jax.ShapeDtypeStruct((N_TOK, D), jnp.float32),
        scratch_types=[
            pltpu.VMEM((BPW,), jnp.int32),
            pltpu.VMEM((BPW, D), jnp.float32),
            pltpu.SemaphoreType.DMA,
        ],
    )
    def gather(table_hbm, idx_hbm, out_hbm, idx_v, rows_v, sem):
        wid = lax.axis_index("s") * SC_NC + lax.axis_index("c")
        base = wid * BPW
        pltpu.sync_copy(idx_hbm.at[pl.ds(base, BPW)], idx_v)
        pltpu.async_copy(table_hbm.at[idx_v], rows_v, sem).wait()
        pltpu.sync_copy(rows_v, out_hbm.at[pl.ds(base, BPW)])

    return gather


def kernel(x, embed):
    flatten = x.reshape(N_TOK, D)
    table = embed[0]
    x_sq = jnp.sum(flatten ** 2, axis=-1, keepdims=True)
    e_sq = jnp.sum(embed ** 2, axis=-1)
    idx = _argmin_call(x_sq, flatten, table, e_sq).reshape(N_TOK)
    quantize = _sc_gather()(table, idx)
    return quantize.reshape(x.shape), idx.reshape(x.shape[:-1])


# speedup vs baseline: 1.3323x; 1.0114x over previous
import functools

import jax
import jax.numpy as jnp
from jax import lax
from jax.experimental import pallas as pl
from jax.experimental.pallas import tpu as pltpu
from jax.experimental.pallas import tpu_sc as plsc

N_TOK = 16 * 576
K = 8192
D = 256

TN = 1152
TK = 4096
NB = N_TOK // TN
KB = K // TK
NSTEP = NB * KB + 1

SC_NC = 2
SC_NS = 16
SC_NW = SC_NC * SC_NS
BPW = N_TOK // SC_NW


def _argmin_body(xsq_ref, x_ref, e_ref, esq_ref, out_ref,
                 x2_s, buf0, buf1, best_val, best_idx):
    t = pl.program_id(0)

    @pl.when(t % KB == 0)
    def _fresh_x():
        x2_s[...] = x_ref[...] + x_ref[...]

    def step(wbuf, rbuf):
        e_tile = e_ref[pl.ds((t % KB) * TK, TK), :]
        wbuf[...] = lax.dot_general(
            x2_s[...], e_tile, (((1,), (1,)), ((), ())),
            preferred_element_type=jnp.float32)
        xsq = xsq_ref[...]
        acc_v = (xsq - rbuf[:, 0:128]) + esq_ref[:, 0:128]
        acc_c = jnp.zeros((TN, 128), jnp.int32)
        for c in range(1, TK // 128):
            sc = (xsq - rbuf[:, c * 128:(c + 1) * 128]) \
                + esq_ref[:, c * 128:(c + 1) * 128]
            mask = sc < acc_v
            acc_v = jnp.where(mask, sc, acc_v)
            acc_c = jnp.where(mask, c, acc_c)
        m = jnp.min(acc_v, axis=1, keepdims=True)
        lane = lax.broadcasted_iota(jnp.int32, (TN, 128), 1)
        gidx = acc_c * 128 + lane
        loc = jnp.min(jnp.where(acc_v == m, gidx, TK), axis=1, keepdims=True)
        cand = loc + ((t - 1) % KB) * TK
        first = (t - 1) % KB == 0
        prev_v = jnp.where(first, jnp.inf, best_val[...])
        prev_i = best_idx[...]
        m = jnp.where(t > 0, m, jnp.inf)
        better = m < prev_v
        best_val[...] = jnp.where(better, m, prev_v)
        best_idx[...] = jnp.where(better, cand, prev_i)

    @pl.when(t % 2 == 0)
    def _even():
        step(buf0, buf1)

    @pl.when(t % 2 == 1)
    def _odd():
        step(buf1, buf0)

    @pl.when((t > 0) & (t % KB == 0))
    def _emit():
        out_ref[...] = best_idx[...]


_argmin_call = pl.pallas_call(
    _argmin_body,
    grid=(NSTEP,),
    in_specs=[
        pl.BlockSpec((TN, 1), lambda t: (jnp.clip((t - 1) // KB, 0, NB - 1), 0)),
        pl.BlockSpec((TN, D), lambda t: (jnp.minimum(t // KB, NB - 1), 0)),
        pl.BlockSpec((K, D), lambda t: (0, 0)),
        pl.BlockSpec((1, TK), lambda t: (0, jnp.maximum(t - 1, 0) % KB)),
    ],
    out_specs=pl.BlockSpec((TN, 1), lambda t: (jnp.clip((t - 1) // KB, 0, NB - 1), 0)),
    out_shape=jax.ShapeDtypeStruct((N_TOK, 1), jnp.int32),
    scratch_shapes=[
        pltpu.VMEM((TN, D), jnp.float32),
        pltpu.VMEM((TN, TK), jnp.float32),
        pltpu.VMEM((TN, TK), jnp.float32),
        pltpu.VMEM((TN, 1), jnp.float32),
        pltpu.VMEM((TN, 1), jnp.int32),
    ],
)


@functools.lru_cache(maxsize=1)
def _sc_gather():
    mesh = plsc.VectorSubcoreMesh(
        core_axis_name="c", subcore_axis_name="s",
        num_cores=SC_NC, num_subcores=SC_NS)

    @functools.partial(
        pl.kernel,
        mesh=mesh,
        out_type=jax.ShapeDtypeStruct((N_TOK, D), jnp.float32),
        scratch_types=[
            pltpu.VMEM((BPW,), jnp.int32),
            pltpu.VMEM((BPW, D), jnp.float32),
            pltpu.SemaphoreType.DMA,
        ],
    )
    def gather(table_hbm, idx_hbm, out_hbm, idx_v, rows_v, sem):
        wid = lax.axis_index("s") * SC_NC + lax.axis_index("c")
        base = wid * BPW
        pltpu.sync_copy(idx_hbm.at[pl.ds(base, BPW)], idx_v)
        pltpu.async_copy(table_hbm.at[idx_v], rows_v, sem).wait()
        pltpu.sync_copy(rows_v, out_hbm.at[pl.ds(base, BPW)])

    return gather


def kernel(x, embed):
    flatten = x.reshape(N_TOK, D)
    table = embed[0]
    x_sq = jnp.sum(flatten ** 2, axis=-1, keepdims=True)
    e_sq = jnp.sum(embed ** 2, axis=-1)
    idx = _argmin_call(x_sq, flatten, table, e_sq).reshape(N_TOK)
    quantize = _sc_gather()(table, idx)
    return quantize.reshape(x.shape), idx.reshape(x.shape[:-1])


# speedup vs baseline: 1.3391x; 1.0050x over previous
import functools

import jax
import jax.numpy as jnp
from jax import lax
from jax.experimental import pallas as pl
from jax.experimental.pallas import tpu as pltpu
from jax.experimental.pallas import tpu_sc as plsc

N_TOK = 16 * 576
K = 8192
D = 256

TN = 1152
TK = 4096
NB = N_TOK // TN
KB = K // TK
NSTEP = NB * KB + 1

SC_NC = 2
SC_NS = 16
SC_NW = SC_NC * SC_NS
BPW = N_TOK // SC_NW


def _argmin_body(xsq_ref, x_ref, e_ref, esq_ref, out_ref,
                 x2_s, buf0, buf1, best_val, best_idx):
    t = pl.program_id(0)

    @pl.when(t % KB == 0)
    def _fresh_x():
        x2_s[...] = x_ref[...] + x_ref[...]

    def step(wbuf, rbuf):
        e_tile = e_ref[pl.ds((t % KB) * TK, TK), :]
        wbuf[...] = lax.dot_general(
            x2_s[...], e_tile, (((1,), (1,)), ((), ())),
            preferred_element_type=jnp.float32)
        xsq = xsq_ref[...]
        acc_v = (xsq - rbuf[:, 0:128]) + esq_ref[:, 0:128]
        acc_c = jnp.zeros((TN, 128), jnp.int32)
        for c in range(1, TK // 128):
            sc = (xsq - rbuf[:, c * 128:(c + 1) * 128]) \
                + esq_ref[:, c * 128:(c + 1) * 128]
            mask = sc < acc_v
            acc_v = jnp.where(mask, sc, acc_v)
            acc_c = jnp.where(mask, c, acc_c)
        m = jnp.min(acc_v, axis=1, keepdims=True)
        lane = lax.broadcasted_iota(jnp.int32, (TN, 128), 1)
        gidx = acc_c * 128 + lane
        loc = jnp.min(jnp.where(acc_v == m, gidx, TK), axis=1, keepdims=True)
        cand = loc + ((t - 1) % KB) * TK
        first = (t - 1) % KB == 0
        prev_v = jnp.where(first, jnp.inf, best_val[...])
        prev_i = best_idx[...]
        m = jnp.where(t > 0, m, jnp.inf)
        better = m < prev_v
        best_val[...] = jnp.where(better, m, prev_v)
        best_idx[...] = jnp.where(better, cand, prev_i)

    @pl.when(t % 2 == 0)
    def _even():
        step(buf0, buf1)

    @pl.when(t % 2 == 1)
    def _odd():
        step(buf1, buf0)

    @pl.when((t > 0) & (t % KB == 0))
    def _emit():
        blk = jnp.clip((t - 1) // KB, 0, NB - 1)
        out_ref[pl.ds(blk * TN, TN)] = best_idx[:, 0]


_argmin_call = pl.pallas_call(
    _argmin_body,
    grid=(NSTEP,),
    in_specs=[
        pl.BlockSpec((TN, 1), lambda t: (jnp.clip((t - 1) // KB, 0, NB - 1), 0)),
        pl.BlockSpec((TN, D), lambda t: (jnp.minimum(t // KB, NB - 1), 0)),
        pl.BlockSpec((K, D), lambda t: (0, 0)),
        pl.BlockSpec((1, TK), lambda t: (0, jnp.maximum(t - 1, 0) % KB)),
    ],
    out_specs=pl.BlockSpec((N_TOK,), lambda t: (0,)),
    out_shape=jax.ShapeDtypeStruct((N_TOK,), jnp.int32),
    scratch_shapes=[
        pltpu.VMEM((TN, D), jnp.float32),
        pltpu.VMEM((TN, TK), jnp.float32),
        pltpu.VMEM((TN, TK), jnp.float32),
        pltpu.VMEM((TN, 1), jnp.float32),
        pltpu.VMEM((TN, 1), jnp.int32),
    ],
)


@functools.lru_cache(maxsize=1)
def _sc_gather():
    mesh = plsc.VectorSubcoreMesh(
        core_axis_name="c", subcore_axis_name="s",
        num_cores=SC_NC, num_subcores=SC_NS)

    @functools.partial(
        pl.kernel,
        mesh=mesh,
        out_type=jax.ShapeDtypeStruct((N_TOK, D), jnp.float32),
        scratch_types=[
            pltpu.VMEM((BPW,), jnp.int32),
            pltpu.VMEM((BPW, D), jnp.float32),
            pltpu.SemaphoreType.DMA,
        ],
    )
    def gather(table_hbm, idx_hbm, out_hbm, idx_v, rows_v, sem):
        wid = lax.axis_index("s") * SC_NC + lax.axis_index("c")
        base = wid * BPW
        pltpu.sync_copy(idx_hbm.at[pl.ds(base, BPW)], idx_v)
        pltpu.async_copy(table_hbm.at[idx_v], rows_v, sem).wait()
        pltpu.sync_copy(rows_v, out_hbm.at[pl.ds(base, BPW)])

    return gather


def kernel(x, embed):
    flatten = x.reshape(N_TOK, D)
    table = embed[0]
    x_sq = jnp.sum(flatten ** 2, axis=-1, keepdims=True)
    e_sq = jnp.sum(embed ** 2, axis=-1)
    idx = _argmin_call(x_sq, flatten, table, e_sq)
    quantize = _sc_gather()(table, idx)
    return quantize.reshape(x.shape), idx.reshape(x.shape[:-1])
